# baseline (speedup 1.0000x reference)
import functools

import jax
import jax.numpy as jnp
from jax import lax
from jax.experimental import pallas as pl
from jax.experimental.pallas import tpu as pltpu
from jax.experimental.pallas import tpu_sc as plsc

N = 10000
E = 320000
D = 128
NC = 2
NS = 16
NW = NC * NS
C = 64
NCH = 160
EPW = NCH * C
EP = NW * EPW
RPT = 632
NPAD = RPT * NS
BR = 1000
GRID = N // BR


def _leaky(v):
    return jnp.where(v > 0, v, 0.1 * v)


@functools.cache
def _get_sc_segsum():
    mesh = plsc.VectorSubcoreMesh(core_axis_name="c", subcore_axis_name="s")
    return functools.partial(
        pl.kernel,
        out_type=(
            jax.ShapeDtypeStruct((NPAD, D), jnp.float32),
            jax.ShapeDtypeStruct((NPAD, D), jnp.float32),
        ),
        mesh=mesh,
        scratch_types=dict(
            src0=pltpu.VMEM((C,), jnp.int32),
            src1=pltpu.VMEM((C,), jnp.int32),
            dst0=pltpu.VMEM((C,), jnp.int32),
            dst1=pltpu.VMEM((C,), jnp.int32),
            rows0=pltpu.VMEM((C, D), jnp.float32),
            rows1=pltpu.VMEM((C, D), jnp.float32),
            acc_sh=pltpu.VMEM_SHARED((NPAD, D), jnp.float32),
            gsem=pltpu.SemaphoreType.DMA,
            ssem=pltpu.SemaphoreType.DMA,
            isem=pltpu.SemaphoreType.DMA,
        ),
    )(_sc_segsum_body)


def _sc_segsum(g, src3, dst3, zeros):
    return _get_sc_segsum()(g, src3, dst3, zeros)


def _sc_segsum_body(g_hbm, src_hbm, dst_hbm, zeros_hbm, out0, out1,
                    src0, src1, dst0, dst1, rows0, rows1, acc_sh,
                    gsem, ssem, isem):
    cid = lax.axis_index("c")
    sid = lax.axis_index("s")
    wid = sid * NC + cid
    rbufs = (rows0, rows1)
    sbufs = (src0, src1)
    dbufs = (dst0, dst1)

    r0 = sid * RPT
    pltpu.sync_copy(zeros_hbm.at[pl.ds(r0, RPT)], acc_sh.at[pl.ds(r0, RPT)])
    pltpu.sync_copy(src_hbm.at[wid, 0], src0)
    pltpu.sync_copy(dst_hbm.at[wid, 0], dst0)
    pltpu.async_copy(src_hbm.at[wid, 1], src1, isem)
    pltpu.async_copy(dst_hbm.at[wid, 1], dst1, isem)
    plsc.subcore_barrier()

    def body(j, carry):
        pltpu.sync_copy(src_hbm.at[wid, j], src0)
        pltpu.sync_copy(dst_hbm.at[wid, j], dst0)
        pltpu.async_copy(g_hbm.at[src0], rows0, gsem).wait()
        pltpu.sync_copy(rows0, acc_sh.at[dst0], add=True)
        return carry

    lax.fori_loop(0, NCH, body, 0)
    plsc.subcore_barrier()

    @pl.when(cid == 0)
    def _():
        pltpu.sync_copy(acc_sh.at[pl.ds(r0, RPT)], out0.at[pl.ds(r0, RPT)])

    @pl.when(cid == 1)
    def _():
        pltpu.sync_copy(acc_sh.at[pl.ds(r0, RPT)], out1.at[pl.ds(r0, RPT)])


def _mm_body(x_ref, w_ref, o_ref):
    o_ref[...] = jnp.dot(x_ref[...], w_ref[...], preferred_element_type=jnp.float32)


def _tc_matmul(x, w):
    return pl.pallas_call(
        _mm_body,
        grid=(GRID,),
        in_specs=[
            pl.BlockSpec((BR, D), lambda i: (i, 0)),
            pl.BlockSpec((D, D), lambda i: (0, 0)),
        ],
        out_specs=pl.BlockSpec((BR, D), lambda i: (i, 0)),
        out_shape=jax.ShapeDtypeStruct((N, D), jnp.float32),
    )(x, w)


def _fuse_body(p0_ref, p1_ref, b_ref, w_ref, h_ref, g_ref):
    h = _leaky(p0_ref[...] + p1_ref[...] + b_ref[...])
    h_ref[...] = h
    g_ref[...] = jnp.dot(h, w_ref[...], preferred_element_type=jnp.float32)


def _tc_fuse(p0, p1, b, w):
    return pl.pallas_call(
        _fuse_body,
        grid=(GRID,),
        in_specs=[
            pl.BlockSpec((BR, D), lambda i: (i, 0)),
            pl.BlockSpec((BR, D), lambda i: (i, 0)),
            pl.BlockSpec((1, D), lambda i: (0, 0)),
            pl.BlockSpec((D, D), lambda i: (0, 0)),
        ],
        out_specs=[
            pl.BlockSpec((BR, D), lambda i: (i, 0)),
            pl.BlockSpec((BR, D), lambda i: (i, 0)),
        ],
        out_shape=[
            jax.ShapeDtypeStruct((N, D), jnp.float32),
            jax.ShapeDtypeStruct((N, D), jnp.float32),
        ],
    )(p0, p1, b.reshape(1, D), w)


def _final_body(p0_ref, p1_ref, b2_ref, x_ref, h1_ref, h2_ref, wo_ref, bo_ref,
                o_ref):
    h3 = _leaky(p0_ref[...] + p1_ref[...] + b2_ref[...])
    wo = wo_ref[...]
    acc = jnp.dot(x_ref[...], wo[0:D], preferred_element_type=jnp.float32)
    acc += jnp.dot(h1_ref[...], wo[D:2 * D], preferred_element_type=jnp.float32)
    acc += jnp.dot(h2_ref[...], wo[2 * D:3 * D], preferred_element_type=jnp.float32)
    acc += jnp.dot(h3, wo[3 * D:4 * D], preferred_element_type=jnp.float32)
    o_ref[...] = _leaky(acc + bo_ref[...])


def _tc_final(p0, p1, b2, x, h1, h2, wout, bout):
    row = pl.BlockSpec((BR, D), lambda i: (i, 0))
    return pl.pallas_call(
        _final_body,
        grid=(GRID,),
        in_specs=[
            row, row,
            pl.BlockSpec((1, D), lambda i: (0, 0)),
            row, row, row,
            pl.BlockSpec((4 * D, D), lambda i: (0, 0)),
            pl.BlockSpec((1, D), lambda i: (0, 0)),
        ],
        out_specs=row,
        out_shape=jax.ShapeDtypeStruct((N, D), jnp.float32),
    )(p0, p1, b2.reshape(1, D), x, h1, h2, wout, bout.reshape(1, D))


def kernel(x, edge_index, W0, b0, W1, b1, W2, b2, Wout, bout):
    pad = EP - E
    src = jnp.concatenate([edge_index[0], jnp.zeros((pad,), jnp.int32)])
    pad_dst = (N + jnp.arange(pad, dtype=jnp.int32) % (NPAD - N)).astype(jnp.int32)
    dst = jnp.concatenate([edge_index[1], pad_dst])
    src = src.reshape(NW, NCH, C)
    dst = dst.reshape(NW, NCH, C)
    zeros = jnp.zeros((NPAD, D), jnp.float32)

    g0 = _tc_matmul(x, W0)
    p0a, p0b = _sc_segsum(g0, src, dst, zeros)
    h1, g1 = _tc_fuse(p0a, p0b, b0, W1)
    p1a, p1b = _sc_segsum(g1, src, dst, zeros)
    h2, g2 = _tc_fuse(p1a, p1b, b1, W2)
    p2a, p2b = _sc_segsum(g2, src, dst, zeros)
    return _tc_final(p2a, p2b, b2, x, h1, h2, Wout, bout)


# speedup vs baseline: 3.4123x; 3.4123x over previous
import functools

import jax
import jax.numpy as jnp
from jax import lax
from jax.experimental import pallas as pl
from jax.experimental.pallas import tpu as pltpu
from jax.experimental.pallas import tpu_sc as plsc

N = 10000
E = 320000
D = 128
NC = 2
NS = 16
NW = NC * NS
C = 80
NCH = 125
EPW = NCH * C
RPT = 632
NPAD = RPT * NS
BR = 1000
GRID = N // BR


def _leaky(v):
    return jnp.where(v > 0, v, 0.1 * v)


@functools.cache
def _get_sc_segsum():
    mesh = plsc.VectorSubcoreMesh(core_axis_name="c", subcore_axis_name="s")
    return functools.partial(
        pl.kernel,
        out_type=(
            jax.ShapeDtypeStruct((NPAD, D), jnp.float32),
            jax.ShapeDtypeStruct((NPAD, D), jnp.float32),
        ),
        mesh=mesh,
        scratch_types=dict(
            src0=pltpu.VMEM((C,), jnp.int32),
            src1=pltpu.VMEM((C,), jnp.int32),
            dst0=pltpu.VMEM((C,), jnp.int32),
            dst1=pltpu.VMEM((C,), jnp.int32),
            rows0=pltpu.VMEM((C, D), jnp.float32),
            rows1=pltpu.VMEM((C, D), jnp.float32),
            acc_sh=pltpu.VMEM_SHARED((NPAD, D), jnp.float32),
            gsem=pltpu.SemaphoreType.DMA,
            ssem=pltpu.SemaphoreType.DMA,
            isem=pltpu.SemaphoreType.DMA,
        ),
    )(_sc_segsum_body)


def _sc_segsum(g, src3, dst3, zeros):
    return _get_sc_segsum()(g, src3, dst3, zeros)


def _sc_segsum_body(g_hbm, src_hbm, dst_hbm, zeros_hbm, out0, out1,
                    src0, src1, dst0, dst1, rows0, rows1, acc_sh,
                    gsem, ssem, isem):
    cid = lax.axis_index("c")
    sid = lax.axis_index("s")
    wid = sid * NC + cid
    rbufs = (rows0, rows1)
    sbufs = (src0, src1)
    dbufs = (dst0, dst1)

    r0 = sid * RPT
    base0 = wid * EPW
    pltpu.sync_copy(zeros_hbm.at[pl.ds(r0, RPT)], acc_sh.at[pl.ds(r0, RPT)])
    pltpu.sync_copy(src_hbm.at[pl.ds(base0, C)], src0)
    pltpu.sync_copy(dst_hbm.at[pl.ds(base0, C)], dst0)
    plsc.subcore_barrier()

    pltpu.async_copy(g_hbm.at[src0], rows0, gsem)

    def outer(o, carry):
        for b in range(2):
            j = o * 2 + b
            cur, nxt = rbufs[b], rbufs[1 - b]

            @pl.when(j < NCH)
            def _():
                @pl.when(j + 1 < NCH)
                def _():
                    base = wid * EPW + (j + 1) * C
                    pltpu.sync_copy(src_hbm.at[pl.ds(base, C)], sbufs[1 - b])
                    pltpu.sync_copy(dst_hbm.at[pl.ds(base, C)], dbufs[1 - b])
                    pltpu.async_copy(g_hbm.at[sbufs[1 - b]], nxt, gsem)

                pltpu.make_async_copy(g_hbm.at[sbufs[b]], cur, gsem).wait()
                pltpu.sync_copy(cur, acc_sh.at[dbufs[b]], add=True)
        return carry

    lax.fori_loop(0, (NCH + 1) // 2, outer, 0)
    plsc.subcore_barrier()

    @pl.when(cid == 0)
    def _():
        pltpu.sync_copy(acc_sh.at[pl.ds(r0, RPT)], out0.at[pl.ds(r0, RPT)])

    @pl.when(cid == 1)
    def _():
        pltpu.sync_copy(acc_sh.at[pl.ds(r0, RPT)], out1.at[pl.ds(r0, RPT)])


def _mm_body(x_ref, w_ref, o_ref):
    o_ref[...] = jnp.dot(x_ref[...], w_ref[...], preferred_element_type=jnp.float32)


def _tc_matmul(x, w):
    return pl.pallas_call(
        _mm_body,
        grid=(GRID,),
        in_specs=[
            pl.BlockSpec((BR, D), lambda i: (i, 0)),
            pl.BlockSpec((D, D), lambda i: (0, 0)),
        ],
        out_specs=pl.BlockSpec((BR, D), lambda i: (i, 0)),
        out_shape=jax.ShapeDtypeStruct((N, D), jnp.float32),
    )(x, w)


def _fuse_body(p0_ref, p1_ref, b_ref, w_ref, h_ref, g_ref):
    h = _leaky(p0_ref[...] + p1_ref[...] + b_ref[...])
    h_ref[...] = h
    g_ref[...] = jnp.dot(h, w_ref[...], preferred_element_type=jnp.float32)


def _tc_fuse(p0, p1, b, w):
    return pl.pallas_call(
        _fuse_body,
        grid=(GRID,),
        in_specs=[
            pl.BlockSpec((BR, D), lambda i: (i, 0)),
            pl.BlockSpec((BR, D), lambda i: (i, 0)),
            pl.BlockSpec((1, D), lambda i: (0, 0)),
            pl.BlockSpec((D, D), lambda i: (0, 0)),
        ],
        out_specs=[
            pl.BlockSpec((BR, D), lambda i: (i, 0)),
            pl.BlockSpec((BR, D), lambda i: (i, 0)),
        ],
        out_shape=[
            jax.ShapeDtypeStruct((N, D), jnp.float32),
            jax.ShapeDtypeStruct((N, D), jnp.float32),
        ],
    )(p0, p1, b.reshape(1, D), w)


def _final_body(p0_ref, p1_ref, b2_ref, x_ref, h1_ref, h2_ref, wo_ref, bo_ref,
                o_ref):
    h3 = _leaky(p0_ref[...] + p1_ref[...] + b2_ref[...])
    wo = wo_ref[...]
    acc = jnp.dot(x_ref[...], wo[0:D], preferred_element_type=jnp.float32)
    acc += jnp.dot(h1_ref[...], wo[D:2 * D], preferred_element_type=jnp.float32)
    acc += jnp.dot(h2_ref[...], wo[2 * D:3 * D], preferred_element_type=jnp.float32)
    acc += jnp.dot(h3, wo[3 * D:4 * D], preferred_element_type=jnp.float32)
    o_ref[...] = _leaky(acc + bo_ref[...])


def _tc_final(p0, p1, b2, x, h1, h2, wout, bout):
    row = pl.BlockSpec((BR, D), lambda i: (i, 0))
    return pl.pallas_call(
        _final_body,
        grid=(GRID,),
        in_specs=[
            row, row,
            pl.BlockSpec((1, D), lambda i: (0, 0)),
            row, row, row,
            pl.BlockSpec((4 * D, D), lambda i: (0, 0)),
            pl.BlockSpec((1, D), lambda i: (0, 0)),
        ],
        out_specs=row,
        out_shape=jax.ShapeDtypeStruct((N, D), jnp.float32),
    )(p0, p1, b2.reshape(1, D), x, h1, h2, wout, bout.reshape(1, D))


def kernel(x, edge_index, W0, b0, W1, b1, W2, b2, Wout, bout):
    src = edge_index[0]
    dst = edge_index[1]
    zeros = jnp.zeros((NPAD, D), jnp.float32)

    g0 = _tc_matmul(x, W0)
    p0a, p0b = _sc_segsum(g0, src, dst, zeros)
    h1, g1 = _tc_fuse(p0a, p0b, b0, W1)
    p1a, p1b = _sc_segsum(g1, src, dst, zeros)
    h2, g2 = _tc_fuse(p1a, p1b, b1, W2)
    p2a, p2b = _sc_segsum(g2, src, dst, zeros)
    return _tc_final(p2a, p2b, b2, x, h1, h2, Wout, bout)


# speedup vs baseline: 5.6724x; 1.6623x over previous
import functools

import jax
import jax.numpy as jnp
from jax import lax
from jax.experimental import pallas as pl
from jax.experimental.pallas import tpu as pltpu
from jax.experimental.pallas import tpu_sc as plsc

N = 10000
E = 320000
D = 128
NC = 2
NS = 16
NW = NC * NS
C = 80
NCH = 125
EPW = NCH * C
RPT = 632
NPAD = RPT * NS
BR = 1000
GRID = N // BR


def _leaky(v):
    return jnp.where(v > 0, v, 0.1 * v)


@functools.cache
def _get_sc_segsum():
    mesh = plsc.VectorSubcoreMesh(core_axis_name="c", subcore_axis_name="s")
    return functools.partial(
        pl.kernel,
        out_type=(
            jax.ShapeDtypeStruct((NPAD, D), jnp.float32),
            jax.ShapeDtypeStruct((NPAD, D), jnp.float32),
        ),
        mesh=mesh,
        scratch_types=dict(
            sbufs=[pltpu.VMEM((C,), jnp.int32)] * 4,
            dbufs=[pltpu.VMEM((C,), jnp.int32)] * 4,
            rbufs=[pltpu.VMEM((C, D), jnp.float32)] * 4,
            acc_sh=pltpu.VMEM_SHARED((NPAD, D), jnp.float32),
            gsem=pltpu.SemaphoreType.DMA,
            isem=pltpu.SemaphoreType.DMA,
        ),
    )(_sc_segsum_body)


def _sc_segsum(g, src3, dst3, zeros):
    return _get_sc_segsum()(g, src3, dst3, zeros)


def _sc_segsum_body(g_hbm, src_hbm, dst_hbm, zeros_hbm, out0, out1,
                    sbufs, dbufs, rbufs, acc_sh, gsem, isem):
    cid = lax.axis_index("c")
    sid = lax.axis_index("s")
    wid = sid * NC + cid

    r0 = sid * RPT
    base0 = wid * EPW
    pltpu.sync_copy(zeros_hbm.at[pl.ds(r0, RPT)], acc_sh.at[pl.ds(r0, RPT)])
    for k in range(2):
        pltpu.sync_copy(src_hbm.at[pl.ds(base0 + k * C, C)], sbufs[k])
        pltpu.sync_copy(dst_hbm.at[pl.ds(base0 + k * C, C)], dbufs[k])
    pltpu.async_copy(src_hbm.at[pl.ds(base0 + 2 * C, C)], sbufs[2], isem)
    pltpu.async_copy(dst_hbm.at[pl.ds(base0 + 2 * C, C)], dbufs[2], isem)
    plsc.subcore_barrier()

    pltpu.async_copy(g_hbm.at[sbufs[0]], rbufs[0], gsem)
    pltpu.async_copy(g_hbm.at[sbufs[1]], rbufs[1], gsem)

    def outer(o, carry):
        for b in range(4):
            j = o * 4 + b

            @pl.when(j < NCH)
            def _():
                pltpu.make_async_copy(g_hbm.at[sbufs[b]], rbufs[b], gsem).wait()

                @pl.when(j + 2 < NCH)
                def _():
                    b2 = (b + 2) % 4
                    pltpu.make_async_copy(
                        src_hbm.at[pl.ds(base0, C)], sbufs[b2], isem).wait()
                    pltpu.make_async_copy(
                        dst_hbm.at[pl.ds(base0, C)], dbufs[b2], isem).wait()
                    pltpu.async_copy(g_hbm.at[sbufs[b2]], rbufs[b2], gsem)

                @pl.when(j + 3 < NCH)
                def _():
                    b3 = (b + 3) % 4
                    base = base0 + (j + 3) * C
                    pltpu.async_copy(src_hbm.at[pl.ds(base, C)], sbufs[b3], isem)
                    pltpu.async_copy(dst_hbm.at[pl.ds(base, C)], dbufs[b3], isem)

                pltpu.sync_copy(rbufs[b], acc_sh.at[dbufs[b]], add=True)
        return carry

    lax.fori_loop(0, (NCH + 3) // 4, outer, 0)
    plsc.subcore_barrier()

    @pl.when(cid == 0)
    def _():
        pltpu.sync_copy(acc_sh.at[pl.ds(r0, RPT)], out0.at[pl.ds(r0, RPT)])

    @pl.when(cid == 1)
    def _():
        pltpu.sync_copy(acc_sh.at[pl.ds(r0, RPT)], out1.at[pl.ds(r0, RPT)])


def _mm_body(x_ref, w_ref, o_ref):
    o_ref[...] = jnp.dot(x_ref[...], w_ref[...], preferred_element_type=jnp.float32)


def _tc_matmul(x, w):
    return pl.pallas_call(
        _mm_body,
        grid=(GRID,),
        in_specs=[
            pl.BlockSpec((BR, D), lambda i: (i, 0)),
            pl.BlockSpec((D, D), lambda i: (0, 0)),
        ],
        out_specs=pl.BlockSpec((BR, D), lambda i: (i, 0)),
        out_shape=jax.ShapeDtypeStruct((N, D), jnp.float32),
    )(x, w)


def _fuse_body(p0_ref, p1_ref, b_ref, w_ref, h_ref, g_ref):
    h = _leaky(p0_ref[...] + p1_ref[...] + b_ref[...])
    h_ref[...] = h
    g_ref[...] = jnp.dot(h, w_ref[...], preferred_element_type=jnp.float32)


def _tc_fuse(p0, p1, b, w):
    return pl.pallas_call(
        _fuse_body,
        grid=(GRID,),
        in_specs=[
            pl.BlockSpec((BR, D), lambda i: (i, 0)),
            pl.BlockSpec((BR, D), lambda i: (i, 0)),
            pl.BlockSpec((1, D), lambda i: (0, 0)),
            pl.BlockSpec((D, D), lambda i: (0, 0)),
        ],
        out_specs=[
            pl.BlockSpec((BR, D), lambda i: (i, 0)),
            pl.BlockSpec((BR, D), lambda i: (i, 0)),
        ],
        out_shape=[
            jax.ShapeDtypeStruct((N, D), jnp.float32),
            jax.ShapeDtypeStruct((N, D), jnp.float32),
        ],
    )(p0, p1, b.reshape(1, D), w)


def _final_body(p0_ref, p1_ref, b2_ref, x_ref, h1_ref, h2_ref, wo_ref, bo_ref,
                o_ref):
    h3 = _leaky(p0_ref[...] + p1_ref[...] + b2_ref[...])
    wo = wo_ref[...]
    acc = jnp.dot(x_ref[...], wo[0:D], preferred_element_type=jnp.float32)
    acc += jnp.dot(h1_ref[...], wo[D:2 * D], preferred_element_type=jnp.float32)
    acc += jnp.dot(h2_ref[...], wo[2 * D:3 * D], preferred_element_type=jnp.float32)
    acc += jnp.dot(h3, wo[3 * D:4 * D], preferred_element_type=jnp.float32)
    o_ref[...] = _leaky(acc + bo_ref[...])


def _tc_final(p0, p1, b2, x, h1, h2, wout, bout):
    row = pl.BlockSpec((BR, D), lambda i: (i, 0))
    return pl.pallas_call(
        _final_body,
        grid=(GRID,),
        in_specs=[
            row, row,
            pl.BlockSpec((1, D), lambda i: (0, 0)),
            row, row, row,
            pl.BlockSpec((4 * D, D), lambda i: (0, 0)),
            pl.BlockSpec((1, D), lambda i: (0, 0)),
        ],
        out_specs=row,
        out_shape=jax.ShapeDtypeStruct((N, D), jnp.float32),
    )(p0, p1, b2.reshape(1, D), x, h1, h2, wout, bout.reshape(1, D))


def kernel(x, edge_index, W0, b0, W1, b1, W2, b2, Wout, bout):
    src = edge_index[0]
    dst = edge_index[1]
    zeros = jnp.zeros((NPAD, D), jnp.float32)

    g0 = _tc_matmul(x, W0)
    p0a, p0b = _sc_segsum(g0, src, dst, zeros)
    h1, g1 = _tc_fuse(p0a, p0b, b0, W1)
    p1a, p1b = _sc_segsum(g1, src, dst, zeros)
    h2, g2 = _tc_fuse(p1a, p1b, b1, W2)
    p2a, p2b = _sc_segsum(g2, src, dst, zeros)
    return _tc_final(p2a, p2b, b2, x, h1, h2, Wout, bout)


# speedup vs baseline: 5.8353x; 1.0287x over previous
import functools

import jax
import jax.numpy as jnp
from jax import lax
from jax.experimental import pallas as pl
from jax.experimental.pallas import tpu as pltpu
from jax.experimental.pallas import tpu_sc as plsc

N = 10000
E = 320000
D = 128
NC = 2
NS = 16
NW = NC * NS
C = 80
NCH = 125
EPW = NCH * C
RPT = 632
NPAD = RPT * NS
BR = 1000
GRID = N // BR


def _leaky(v):
    return jnp.where(v > 0, v, 0.1 * v)


@functools.cache
def _get_sc_segsum():
    mesh = plsc.VectorSubcoreMesh(core_axis_name="c", subcore_axis_name="s")
    return functools.partial(
        pl.kernel,
        out_type=(
            jax.ShapeDtypeStruct((NPAD, D), jnp.float32),
            jax.ShapeDtypeStruct((NPAD, D), jnp.float32),
        ),
        mesh=mesh,
        scratch_types=dict(
            sbufs=[pltpu.VMEM((C,), jnp.int32)] * 8,
            dbufs=[pltpu.VMEM((C,), jnp.int32)] * 8,
            rbufs=[pltpu.VMEM((C, D), jnp.float32)] * 4,
            acc_sh=pltpu.VMEM_SHARED((NPAD, D), jnp.float32),
            gsem=pltpu.SemaphoreType.DMA,
            isem=pltpu.SemaphoreType.DMA,
        ),
    )(_sc_segsum_body)


def _sc_segsum(g, src3, dst3, zeros):
    return _get_sc_segsum()(g, src3, dst3, zeros)


def _sc_segsum_body(g_hbm, src_hbm, dst_hbm, zeros_hbm, out0, out1,
                    sbufs, dbufs, rbufs, acc_sh, gsem, isem):
    cid = lax.axis_index("c")
    sid = lax.axis_index("s")
    wid = sid * NC + cid

    r0 = sid * RPT
    base0 = wid * EPW
    pltpu.sync_copy(zeros_hbm.at[pl.ds(r0, RPT)], acc_sh.at[pl.ds(r0, RPT)])
    for k in range(3):
        pltpu.sync_copy(src_hbm.at[pl.ds(base0 + k * C, C)], sbufs[k])
        pltpu.sync_copy(dst_hbm.at[pl.ds(base0 + k * C, C)], dbufs[k])
    pltpu.async_copy(src_hbm.at[pl.ds(base0 + 3 * C, C)], sbufs[3], isem)
    pltpu.async_copy(dst_hbm.at[pl.ds(base0 + 3 * C, C)], dbufs[3], isem)
    plsc.subcore_barrier()

    for k in range(3):
        pltpu.async_copy(g_hbm.at[sbufs[k]], rbufs[k], gsem)

    def outer(o, carry):
        for b in range(8):
            j = o * 8 + b

            @pl.when(j < NCH)
            def _():
                pltpu.make_async_copy(
                    g_hbm.at[sbufs[b]], rbufs[b % 4], gsem).wait()

                @pl.when(j + 3 < NCH)
                def _():
                    b3 = (b + 3) % 8
                    pltpu.make_async_copy(
                        src_hbm.at[pl.ds(base0, C)], sbufs[b3], isem).wait()
                    pltpu.make_async_copy(
                        dst_hbm.at[pl.ds(base0, C)], dbufs[b3], isem).wait()
                    pltpu.async_copy(g_hbm.at[sbufs[b3]], rbufs[(b + 3) % 4],
                                     gsem)

                @pl.when(j + 4 < NCH)
                def _():
                    b4 = (b + 4) % 8
                    base = base0 + (j + 4) * C
                    pltpu.async_copy(src_hbm.at[pl.ds(base, C)], sbufs[b4], isem)
                    pltpu.async_copy(dst_hbm.at[pl.ds(base, C)], dbufs[b4], isem)

                pltpu.sync_copy(rbufs[b % 4], acc_sh.at[dbufs[b]], add=True)
        return carry

    lax.fori_loop(0, (NCH + 7) // 8, outer, 0)
    plsc.subcore_barrier()

    @pl.when(cid == 0)
    def _():
        pltpu.sync_copy(acc_sh.at[pl.ds(r0, RPT)], out0.at[pl.ds(r0, RPT)])

    @pl.when(cid == 1)
    def _():
        pltpu.sync_copy(acc_sh.at[pl.ds(r0, RPT)], out1.at[pl.ds(r0, RPT)])


def _mm_body(x_ref, w_ref, o_ref):
    o_ref[...] = jnp.dot(x_ref[...], w_ref[...], preferred_element_type=jnp.float32)


def _tc_matmul(x, w):
    return pl.pallas_call(
        _mm_body,
        grid=(GRID,),
        in_specs=[
            pl.BlockSpec((BR, D), lambda i: (i, 0)),
            pl.BlockSpec((D, D), lambda i: (0, 0)),
        ],
        out_specs=pl.BlockSpec((BR, D), lambda i: (i, 0)),
        out_shape=jax.ShapeDtypeStruct((N, D), jnp.float32),
    )(x, w)


def _fuse_body(p0_ref, p1_ref, b_ref, w_ref, h_ref, g_ref):
    h = _leaky(p0_ref[...] + p1_ref[...] + b_ref[...])
    h_ref[...] = h
    g_ref[...] = jnp.dot(h, w_ref[...], preferred_element_type=jnp.float32)


def _tc_fuse(p0, p1, b, w):
    return pl.pallas_call(
        _fuse_body,
        grid=(GRID,),
        in_specs=[
            pl.BlockSpec((BR, D), lambda i: (i, 0)),
            pl.BlockSpec((BR, D), lambda i: (i, 0)),
            pl.BlockSpec((1, D), lambda i: (0, 0)),
            pl.BlockSpec((D, D), lambda i: (0, 0)),
        ],
        out_specs=[
            pl.BlockSpec((BR, D), lambda i: (i, 0)),
            pl.BlockSpec((BR, D), lambda i: (i, 0)),
        ],
        out_shape=[
            jax.ShapeDtypeStruct((N, D), jnp.float32),
            jax.ShapeDtypeStruct((N, D), jnp.float32),
        ],
    )(p0, p1, b.reshape(1, D), w)


def _final_body(p0_ref, p1_ref, b2_ref, x_ref, h1_ref, h2_ref, wo_ref, bo_ref,
                o_ref):
    h3 = _leaky(p0_ref[...] + p1_ref[...] + b2_ref[...])
    wo = wo_ref[...]
    acc = jnp.dot(x_ref[...], wo[0:D], preferred_element_type=jnp.float32)
    acc += jnp.dot(h1_ref[...], wo[D:2 * D], preferred_element_type=jnp.float32)
    acc += jnp.dot(h2_ref[...], wo[2 * D:3 * D], preferred_element_type=jnp.float32)
    acc += jnp.dot(h3, wo[3 * D:4 * D], preferred_element_type=jnp.float32)
    o_ref[...] = _leaky(acc + bo_ref[...])


def _tc_final(p0, p1, b2, x, h1, h2, wout, bout):
    row = pl.BlockSpec((BR, D), lambda i: (i, 0))
    return pl.pallas_call(
        _final_body,
        grid=(GRID,),
        in_specs=[
            row, row,
            pl.BlockSpec((1, D), lambda i: (0, 0)),
            row, row, row,
            pl.BlockSpec((4 * D, D), lambda i: (0, 0)),
            pl.BlockSpec((1, D), lambda i: (0, 0)),
        ],
        out_specs=row,
        out_shape=jax.ShapeDtypeStruct((N, D), jnp.float32),
    )(p0, p1, b2.reshape(1, D), x, h1, h2, wout, bout.reshape(1, D))


def kernel(x, edge_index, W0, b0, W1, b1, W2, b2, Wout, bout):
    src = edge_index[0]
    dst = edge_index[1]
    zeros = jnp.zeros((NPAD, D), jnp.float32)

    g0 = _tc_matmul(x, W0)
    p0a, p0b = _sc_segsum(g0, src, dst, zeros)
    h1, g1 = _tc_fuse(p0a, p0b, b0, W1)
    p1a, p1b = _sc_segsum(g1, src, dst, zeros)
    h2, g2 = _tc_fuse(p1a, p1b, b1, W2)
    p2a, p2b = _sc_segsum(g2, src, dst, zeros)
    return _tc_final(p2a, p2b, b2, x, h1, h2, Wout, bout)


# speedup vs baseline: 5.8413x; 1.0010x over previous
import functools

import jax
import jax.numpy as jnp
from jax import lax
from jax.experimental import pallas as pl
from jax.experimental.pallas import tpu as pltpu
from jax.experimental.pallas import tpu_sc as plsc

N = 10000
E = 320000
D = 128
NC = 2
NS = 16
NW = NC * NS
C = 80
NCH = 125
EPW = NCH * C
RPT = 632
NPAD = RPT * NS
BR = 1000
GRID = N // BR


def _leaky(v):
    return jnp.where(v > 0, v, 0.1 * v)


@functools.cache
def _get_sc_segsum():
    mesh = plsc.VectorSubcoreMesh(core_axis_name="c", subcore_axis_name="s")
    return functools.partial(
        pl.kernel,
        out_type=(
            jax.ShapeDtypeStruct((NPAD, D), jnp.float32),
            jax.ShapeDtypeStruct((NPAD, D), jnp.float32),
        ),
        mesh=mesh,
        scratch_types=dict(
            sbufs=[pltpu.VMEM((C,), jnp.int32)] * 8,
            dbufs=[pltpu.VMEM((C,), jnp.int32)] * 8,
            rbufs=[pltpu.VMEM((C, D), jnp.float32)] * 4,
            acc_sh=pltpu.VMEM_SHARED((NPAD, D), jnp.float32),
            gsem=pltpu.SemaphoreType.DMA,
            isem=pltpu.SemaphoreType.DMA,
            ssem=pltpu.SemaphoreType.DMA,
        ),
    )(_sc_segsum_body)


def _sc_segsum(g, src3, dst3, zeros):
    return _get_sc_segsum()(g, src3, dst3, zeros)


def _sc_segsum_body(g_hbm, src_hbm, dst_hbm, zeros_hbm, out0, out1,
                    sbufs, dbufs, rbufs, acc_sh, gsem, isem, ssem):
    cid = lax.axis_index("c")
    sid = lax.axis_index("s")
    wid = sid * NC + cid

    r0 = sid * RPT
    base0 = wid * EPW
    pltpu.sync_copy(zeros_hbm.at[pl.ds(r0, RPT)], acc_sh.at[pl.ds(r0, RPT)])
    for k in range(3):
        pltpu.sync_copy(src_hbm.at[pl.ds(base0 + k * C, C)], sbufs[k])
        pltpu.sync_copy(dst_hbm.at[pl.ds(base0 + k * C, C)], dbufs[k])
    pltpu.async_copy(src_hbm.at[pl.ds(base0 + 3 * C, C)], sbufs[3], isem)
    pltpu.async_copy(dst_hbm.at[pl.ds(base0 + 3 * C, C)], dbufs[3], isem)
    plsc.subcore_barrier()

    for k in range(3):
        pltpu.async_copy(g_hbm.at[sbufs[k]], rbufs[k], gsem)

    def outer(o, carry):
        for b in range(8):
            j = o * 8 + b

            @pl.when(j < NCH)
            def _():
                pltpu.make_async_copy(
                    g_hbm.at[sbufs[b]], rbufs[b % 4], gsem).wait()

                @pl.when(j + 3 < NCH)
                def _():
                    b3 = (b + 3) % 8
                    pltpu.make_async_copy(
                        src_hbm.at[pl.ds(base0, C)], sbufs[b3], isem).wait()
                    pltpu.make_async_copy(
                        dst_hbm.at[pl.ds(base0, C)], dbufs[b3], isem).wait()

                    @pl.when(j > 0)
                    def _():
                        pltpu.make_async_copy(
                            rbufs[(b + 3) % 4], acc_sh.at[dbufs[b]],
                            ssem).wait()

                    pltpu.async_copy(g_hbm.at[sbufs[b3]], rbufs[(b + 3) % 4],
                                     gsem)

                @pl.when(j + 4 < NCH)
                def _():
                    b4 = (b + 4) % 8
                    base = base0 + (j + 4) * C
                    pltpu.async_copy(src_hbm.at[pl.ds(base, C)], sbufs[b4], isem)
                    pltpu.async_copy(dst_hbm.at[pl.ds(base, C)], dbufs[b4], isem)

                pltpu.async_copy(rbufs[b % 4], acc_sh.at[dbufs[b]], ssem,
                                 add=True)
        return carry

    lax.fori_loop(0, (NCH + 7) // 8, outer, 0)
    for _ in range(4):
        pltpu.make_async_copy(rbufs[0], acc_sh.at[dbufs[0]], ssem).wait()
    plsc.subcore_barrier()

    @pl.when(cid == 0)
    def _():
        pltpu.sync_copy(acc_sh.at[pl.ds(r0, RPT)], out0.at[pl.ds(r0, RPT)])

    @pl.when(cid == 1)
    def _():
        pltpu.sync_copy(acc_sh.at[pl.ds(r0, RPT)], out1.at[pl.ds(r0, RPT)])


def _mm_body(x_ref, w_ref, o_ref):
    o_ref[...] = jnp.dot(x_ref[...], w_ref[...], preferred_element_type=jnp.float32)


def _tc_matmul(x, w):
    return pl.pallas_call(
        _mm_body,
        grid=(GRID,),
        in_specs=[
            pl.BlockSpec((BR, D), lambda i: (i, 0)),
            pl.BlockSpec((D, D), lambda i: (0, 0)),
        ],
        out_specs=pl.BlockSpec((BR, D), lambda i: (i, 0)),
        out_shape=jax.ShapeDtypeStruct((N, D), jnp.float32),
    )(x, w)


def _fuse_body(p0_ref, p1_ref, b_ref, w_ref, h_ref, g_ref):
    h = _leaky(p0_ref[...] + p1_ref[...] + b_ref[...])
    h_ref[...] = h
    g_ref[...] = jnp.dot(h, w_ref[...], preferred_element_type=jnp.float32)


def _tc_fuse(p0, p1, b, w):
    return pl.pallas_call(
        _fuse_body,
        grid=(GRID,),
        in_specs=[
            pl.BlockSpec((BR, D), lambda i: (i, 0)),
            pl.BlockSpec((BR, D), lambda i: (i, 0)),
            pl.BlockSpec((1, D), lambda i: (0, 0)),
            pl.BlockSpec((D, D), lambda i: (0, 0)),
        ],
        out_specs=[
            pl.BlockSpec((BR, D), lambda i: (i, 0)),
            pl.BlockSpec((BR, D), lambda i: (i, 0)),
        ],
        out_shape=[
            jax.ShapeDtypeStruct((N, D), jnp.float32),
            jax.ShapeDtypeStruct((N, D), jnp.float32),
        ],
    )(p0, p1, b.reshape(1, D), w)


def _final_body(p0_ref, p1_ref, b2_ref, x_ref, h1_ref, h2_ref, wo_ref, bo_ref,
                o_ref):
    h3 = _leaky(p0_ref[...] + p1_ref[...] + b2_ref[...])
    wo = wo_ref[...]
    acc = jnp.dot(x_ref[...], wo[0:D], preferred_element_type=jnp.float32)
    acc += jnp.dot(h1_ref[...], wo[D:2 * D], preferred_element_type=jnp.float32)
    acc += jnp.dot(h2_ref[...], wo[2 * D:3 * D], preferred_element_type=jnp.float32)
    acc += jnp.dot(h3, wo[3 * D:4 * D], preferred_element_type=jnp.float32)
    o_ref[...] = _leaky(acc + bo_ref[...])


def _tc_final(p0, p1, b2, x, h1, h2, wout, bout):
    row = pl.BlockSpec((BR, D), lambda i: (i, 0))
    return pl.pallas_call(
        _final_body,
        grid=(GRID,),
        in_specs=[
            row, row,
            pl.BlockSpec((1, D), lambda i: (0, 0)),
            row, row, row,
            pl.BlockSpec((4 * D, D), lambda i: (0, 0)),
            pl.BlockSpec((1, D), lambda i: (0, 0)),
        ],
        out_specs=row,
        out_shape=jax.ShapeDtypeStruct((N, D), jnp.float32),
    )(p0, p1, b2.reshape(1, D), x, h1, h2, wout, bout.reshape(1, D))


def kernel(x, edge_index, W0, b0, W1, b1, W2, b2, Wout, bout):
    src = edge_index[0]
    dst = edge_index[1]
    zeros = jnp.zeros((NPAD, D), jnp.float32)

    g0 = _tc_matmul(x, W0)
    p0a, p0b = _sc_segsum(g0, src, dst, zeros)
    h1, g1 = _tc_fuse(p0a, p0b, b0, W1)
    p1a, p1b = _sc_segsum(g1, src, dst, zeros)
    h2, g2 = _tc_fuse(p1a, p1b, b1, W2)
    p2a, p2b = _sc_segsum(g2, src, dst, zeros)
    return _tc_final(p2a, p2b, b2, x, h1, h2, Wout, bout)


# speedup vs baseline: 6.0856x; 1.0418x over previous
import functools

import jax
import jax.numpy as jnp
from jax import lax
from jax.experimental import pallas as pl
from jax.experimental.pallas import tpu as pltpu
from jax.experimental.pallas import tpu_sc as plsc

N = 10000
E = 320000
D = 128
NC = 2
NS = 16
NW = NC * NS
C = 80
NCH = 125
EPW = NCH * C
RPT = 632
NPAD = RPT * NS
BR = 1000
GRID = N // BR


def _leaky(v):
    return jnp.where(v > 0, v, 0.1 * v)


@functools.cache
def _get_sc_segsum():
    mesh = plsc.VectorSubcoreMesh(core_axis_name="c", subcore_axis_name="s")
    return functools.partial(
        pl.kernel,
        out_type=(
            jax.ShapeDtypeStruct((NPAD, D), jnp.float32),
            jax.ShapeDtypeStruct((NPAD, D), jnp.float32),
        ),
        mesh=mesh,
        scratch_types=dict(
            sbufs=[pltpu.VMEM((C,), jnp.int32)] * 8,
            dbufs=[pltpu.VMEM((C,), jnp.int32)] * 8,
            rbufs=[pltpu.VMEM((C, D), jnp.float32)] * 4,
            acc_sh=pltpu.VMEM_SHARED((NPAD, D), jnp.float32),
            gsem=pltpu.SemaphoreType.DMA,
            isem=pltpu.SemaphoreType.DMA,
            ssem=pltpu.SemaphoreType.DMA,
        ),
    )(_sc_segsum_body)


def _sc_segsum(g, ei, zeros):
    return _get_sc_segsum()(g, ei, zeros)


def _sc_segsum_body(g_hbm, ei_hbm, zeros_hbm, out0, out1,
                    sbufs, dbufs, rbufs, acc_sh, gsem, isem, ssem):
    cid = lax.axis_index("c")
    sid = lax.axis_index("s")
    wid = sid * NC + cid

    r0 = sid * RPT
    base0 = wid * EPW
    for k in range(3):
        pltpu.sync_copy(ei_hbm.at[pl.ds(base0 + k * C, C)], sbufs[k])
        pltpu.sync_copy(ei_hbm.at[pl.ds(E + base0 + k * C, C)], dbufs[k])
    pltpu.async_copy(ei_hbm.at[pl.ds(base0 + 3 * C, C)], sbufs[3], isem)
    pltpu.async_copy(ei_hbm.at[pl.ds(E + base0 + 3 * C, C)], dbufs[3], isem)
    for k in range(3):
        pltpu.async_copy(g_hbm.at[sbufs[k]], rbufs[k], gsem)
    pltpu.sync_copy(zeros_hbm.at[pl.ds(r0, RPT)], acc_sh.at[pl.ds(r0, RPT)])
    plsc.subcore_barrier()

    def outer(o, carry):
        for b in range(8):
            j = o * 8 + b

            @pl.when(j < NCH)
            def _():
                pltpu.make_async_copy(
                    g_hbm.at[sbufs[b]], rbufs[b % 4], gsem).wait()

                @pl.when(j + 3 < NCH)
                def _():
                    b3 = (b + 3) % 8
                    pltpu.make_async_copy(
                        ei_hbm.at[pl.ds(base0, C)], sbufs[b3], isem).wait()
                    pltpu.make_async_copy(
                        ei_hbm.at[pl.ds(base0, C)], dbufs[b3], isem).wait()

                    @pl.when(j > 0)
                    def _():
                        pltpu.make_async_copy(
                            rbufs[(b + 3) % 4], acc_sh.at[dbufs[b]],
                            ssem).wait()

                    pltpu.async_copy(g_hbm.at[sbufs[b3]], rbufs[(b + 3) % 4],
                                     gsem)

                @pl.when(j + 4 < NCH)
                def _():
                    b4 = (b + 4) % 8
                    base = base0 + (j + 4) * C
                    pltpu.async_copy(ei_hbm.at[pl.ds(base, C)], sbufs[b4], isem)
                    pltpu.async_copy(ei_hbm.at[pl.ds(E + base, C)], dbufs[b4],
                                     isem)

                pltpu.async_copy(rbufs[b % 4], acc_sh.at[dbufs[b]], ssem,
                                 add=True)
        return carry

    lax.fori_loop(0, (NCH + 7) // 8, outer, 0)
    for _ in range(4):
        pltpu.make_async_copy(rbufs[0], acc_sh.at[dbufs[0]], ssem).wait()
    plsc.subcore_barrier()

    @pl.when(cid == 0)
    def _():
        pltpu.sync_copy(acc_sh.at[pl.ds(r0, RPT)], out0.at[pl.ds(r0, RPT)])

    @pl.when(cid == 1)
    def _():
        pltpu.sync_copy(acc_sh.at[pl.ds(r0, RPT)], out1.at[pl.ds(r0, RPT)])


def _mm_body(x_ref, w_ref, o_ref):
    o_ref[...] = jnp.dot(x_ref[...], w_ref[...], preferred_element_type=jnp.float32)


def _tc_matmul(x, w):
    return pl.pallas_call(
        _mm_body,
        grid=(GRID,),
        in_specs=[
            pl.BlockSpec((BR, D), lambda i: (i, 0)),
            pl.BlockSpec((D, D), lambda i: (0, 0)),
        ],
        out_specs=pl.BlockSpec((BR, D), lambda i: (i, 0)),
        out_shape=jax.ShapeDtypeStruct((N, D), jnp.float32),
    )(x, w)


def _fuse_body(p0_ref, p1_ref, b_ref, w_ref, h_ref, g_ref):
    h = _leaky(p0_ref[...] + p1_ref[...] + b_ref[...])
    h_ref[...] = h
    g_ref[...] = jnp.dot(h, w_ref[...], preferred_element_type=jnp.float32)


def _tc_fuse(p0, p1, b, w):
    return pl.pallas_call(
        _fuse_body,
        grid=(GRID,),
        in_specs=[
            pl.BlockSpec((BR, D), lambda i: (i, 0)),
            pl.BlockSpec((BR, D), lambda i: (i, 0)),
            pl.BlockSpec((1, D), lambda i: (0, 0)),
            pl.BlockSpec((D, D), lambda i: (0, 0)),
        ],
        out_specs=[
            pl.BlockSpec((BR, D), lambda i: (i, 0)),
            pl.BlockSpec((BR, D), lambda i: (i, 0)),
        ],
        out_shape=[
            jax.ShapeDtypeStruct((N, D), jnp.float32),
            jax.ShapeDtypeStruct((N, D), jnp.float32),
        ],
    )(p0, p1, b.reshape(1, D), w)


def _final_body(p0_ref, p1_ref, b2_ref, x_ref, h1_ref, h2_ref, wo_ref, bo_ref,
                o_ref):
    h3 = _leaky(p0_ref[...] + p1_ref[...] + b2_ref[...])
    wo = wo_ref[...]
    acc = jnp.dot(x_ref[...], wo[0:D], preferred_element_type=jnp.float32)
    acc += jnp.dot(h1_ref[...], wo[D:2 * D], preferred_element_type=jnp.float32)
    acc += jnp.dot(h2_ref[...], wo[2 * D:3 * D], preferred_element_type=jnp.float32)
    acc += jnp.dot(h3, wo[3 * D:4 * D], preferred_element_type=jnp.float32)
    o_ref[...] = _leaky(acc + bo_ref[...])


def _tc_final(p0, p1, b2, x, h1, h2, wout, bout):
    row = pl.BlockSpec((BR, D), lambda i: (i, 0))
    return pl.pallas_call(
        _final_body,
        grid=(GRID,),
        in_specs=[
            row, row,
            pl.BlockSpec((1, D), lambda i: (0, 0)),
            row, row, row,
            pl.BlockSpec((4 * D, D), lambda i: (0, 0)),
            pl.BlockSpec((1, D), lambda i: (0, 0)),
        ],
        out_specs=row,
        out_shape=jax.ShapeDtypeStruct((N, D), jnp.float32),
    )(p0, p1, b2.reshape(1, D), x, h1, h2, wout, bout.reshape(1, D))


def kernel(x, edge_index, W0, b0, W1, b1, W2, b2, Wout, bout):
    ei = edge_index.reshape(2 * E)
    zeros = jnp.zeros((NPAD, D), jnp.float32)

    g0 = _tc_matmul(x, W0)
    p0a, p0b = _sc_segsum(g0, ei, zeros)
    h1, g1 = _tc_fuse(p0a, p0b, b0, W1)
    p1a, p1b = _sc_segsum(g1, ei, zeros)
    h2, g2 = _tc_fuse(p1a, p1b, b1, W2)
    p2a, p2b = _sc_segsum(g2, ei, zeros)
    return _tc_final(p2a, p2b, b2, x, h1, h2, Wout, bout)
